# baseline (speedup 1.0000x reference)
import functools

import jax
import jax.numpy as jnp
from jax import lax
from jax.experimental import pallas as pl
from jax.experimental.pallas import tpu as pltpu
from jax.experimental.pallas import tpu_sc as plsc

_NC = 2
_NS = 16
_K = 128
_SPLIT0 = 0.25


def _sc_aggregate(x, src, dst, zeros, *, n_pad, rps, bpw0, bpw1):
    d = x.shape[1]
    mesh = plsc.VectorSubcoreMesh(core_axis_name="c", subcore_axis_name="s")

    seg_max = 80

    @functools.partial(
        pl.kernel,
        out_type=jax.ShapeDtypeStruct((_NC * n_pad, d), jnp.float32),
        mesh=mesh,
        scratch_types=[
            pltpu.VMEM((seg_max, 1, _K), jnp.int32),
            pltpu.VMEM((seg_max, 1, _K), jnp.int32),
            pltpu.VMEM((_K, d), jnp.float32),
            pltpu.VMEM_SHARED((n_pad, d), jnp.float32),
        ],
    )
    def agg_kernel(x_hbm, src_hbm, dst_hbm, z_hbm, out_hbm,
                   sidx, didx, rows, acc):
        cid = lax.axis_index("c")
        sid = lax.axis_index("s")

        pltpu.sync_copy(z_hbm, acc.at[pl.ds(sid * rps, rps)])
        plsc.subcore_barrier()

        def emit_worker(base, nblk):
            off = 0
            while off < nblk:
                seg = min(seg_max, nblk - off)
                pltpu.sync_copy(src_hbm.at[pl.ds(base + off, seg)],
                                sidx.at[pl.ds(0, seg)])
                pltpu.sync_copy(dst_hbm.at[pl.ds(base + off, seg)],
                                didx.at[pl.ds(0, seg)])
                for j in range(seg):
                    pltpu.sync_copy(x_hbm.at[sidx.at[j, 0]], rows)
                    pltpu.sync_copy(rows, acc.at[didx.at[j, 0]], add=True)
                off += seg

        @pl.when(cid == 0)
        def _():
            emit_worker(sid * bpw0, bpw0)

        @pl.when(cid == 1)
        def _():
            emit_worker(_NS * bpw0 + sid * bpw1, bpw1)

        plsc.subcore_barrier()
        pltpu.sync_copy(
            acc.at[pl.ds(sid * rps, rps)],
            out_hbm.at[pl.ds(cid * n_pad + sid * rps, rps)],
        )

    return agg_kernel(x, src, dst, zeros)


def _mlp(x, p0, p1, W1, b1, W2, b2):
    n, d = x.shape
    r = 1000
    assert n % r == 0

    def body(x_ref, p0_ref, p1_ref, w1_ref, b1_ref, w2_ref, b2_ref, o_ref):
        h = x_ref[...] + p0_ref[...] + p1_ref[...]
        h = jnp.dot(h, w1_ref[...], preferred_element_type=jnp.float32)
        h = jnp.maximum(h + b1_ref[...], 0.0)
        o = jnp.dot(h, w2_ref[...], preferred_element_type=jnp.float32)
        o_ref[...] = o + b2_ref[...]

    return pl.pallas_call(
        body,
        grid=(n // r,),
        in_specs=[
            pl.BlockSpec((r, d), lambda i: (i, 0)),
            pl.BlockSpec((r, d), lambda i: (i, 0)),
            pl.BlockSpec((r, d), lambda i: (i, 0)),
            pl.BlockSpec((d, d), lambda i: (0, 0)),
            pl.BlockSpec((1, d), lambda i: (0, 0)),
            pl.BlockSpec((d, d), lambda i: (0, 0)),
            pl.BlockSpec((1, d), lambda i: (0, 0)),
        ],
        out_specs=pl.BlockSpec((r, d), lambda i: (i, 0)),
        out_shape=jax.ShapeDtypeStruct((n, d), jnp.float32),
    )(x, p0, p1, W1, b1.reshape(1, d), W2, b2.reshape(1, d))


def kernel(x, edge_index, W1, b1, W2, b2):
    n, d = x.shape
    e = edge_index.shape[1]

    rps = -(-(n + 1) // _NS)
    rps = -(-rps // 8) * 8
    n_pad = _NS * rps

    blocks_pad = -(-e // (_K * _NS)) * _NS
    e_pad = blocks_pad * _K
    bpw_tot = blocks_pad // _NS
    bpw0 = max(1, round(bpw_tot * _SPLIT0))
    bpw1 = bpw_tot - bpw0
    src = edge_index[0]
    dst = edge_index[1]
    if e_pad != e:
        pad = e_pad - e
        src = jnp.concatenate([src, jnp.zeros((pad,), jnp.int32)])
        dst = jnp.concatenate([dst, jnp.full((pad,), n, jnp.int32)])
    src = src.reshape(blocks_pad, 1, _K)
    dst = dst.reshape(blocks_pad, 1, _K)

    zeros = jnp.zeros((rps, d), jnp.float32)
    partials = _sc_aggregate(x, src, dst, zeros, n_pad=n_pad, rps=rps,
                             bpw0=bpw0, bpw1=bpw1)
    p0 = partials[:n]
    p1 = partials[n_pad:n_pad + n]
    return _mlp(x, p0, p1, W1, b1, W2, b2)


# speedup vs baseline: 1.1318x; 1.1318x over previous
import functools

import jax
import jax.numpy as jnp
from jax import lax
from jax.experimental import pallas as pl
from jax.experimental.pallas import tpu as pltpu
from jax.experimental.pallas import tpu_sc as plsc

_NC = 2
_NS = 16
_K = 128
_SPLIT0 = 0.40


def _sc_aggregate(x, src, dst, zeros, *, n_pad, rps, bpw0, bpw1):
    d = x.shape[1]
    mesh = plsc.VectorSubcoreMesh(core_axis_name="c", subcore_axis_name="s")

    seg_max = 80

    @functools.partial(
        pl.kernel,
        out_type=jax.ShapeDtypeStruct((_NC * n_pad, d), jnp.float32),
        mesh=mesh,
        scratch_types=[
            pltpu.VMEM((seg_max, 1, _K), jnp.int32),
            pltpu.VMEM((seg_max, 1, _K), jnp.int32),
            pltpu.VMEM((_K, d), jnp.float32),
            pltpu.VMEM_SHARED((n_pad, d), jnp.float32),
        ],
    )
    def agg_kernel(x_hbm, src_hbm, dst_hbm, z_hbm, out_hbm,
                   sidx, didx, rows, acc):
        cid = lax.axis_index("c")
        sid = lax.axis_index("s")

        pltpu.sync_copy(z_hbm, acc.at[pl.ds(sid * rps, rps)])
        plsc.subcore_barrier()

        def emit_worker(base, nblk):
            off = 0
            while off < nblk:
                seg = min(seg_max, nblk - off)
                pltpu.sync_copy(src_hbm.at[pl.ds(base + off, seg)],
                                sidx.at[pl.ds(0, seg)])
                pltpu.sync_copy(dst_hbm.at[pl.ds(base + off, seg)],
                                didx.at[pl.ds(0, seg)])
                for j in range(seg):
                    pltpu.sync_copy(x_hbm.at[sidx.at[j, 0]], rows)
                    pltpu.sync_copy(rows, acc.at[didx.at[j, 0]], add=True)
                off += seg

        @pl.when(cid == 0)
        def _():
            emit_worker(sid * bpw0, bpw0)

        @pl.when(cid == 1)
        def _():
            emit_worker(_NS * bpw0 + sid * bpw1, bpw1)

        plsc.subcore_barrier()
        pltpu.sync_copy(
            acc.at[pl.ds(sid * rps, rps)],
            out_hbm.at[pl.ds(cid * n_pad + sid * rps, rps)],
        )

    return agg_kernel(x, src, dst, zeros)


def _mlp(x, p0, p1, W1, b1, W2, b2):
    n, d = x.shape
    r = 1000
    assert n % r == 0

    def body(x_ref, p0_ref, p1_ref, w1_ref, b1_ref, w2_ref, b2_ref, o_ref):
        h = x_ref[...] + p0_ref[...] + p1_ref[...]
        h = jnp.dot(h, w1_ref[...], preferred_element_type=jnp.float32)
        h = jnp.maximum(h + b1_ref[...], 0.0)
        o = jnp.dot(h, w2_ref[...], preferred_element_type=jnp.float32)
        o_ref[...] = o + b2_ref[...]

    return pl.pallas_call(
        body,
        grid=(n // r,),
        in_specs=[
            pl.BlockSpec((r, d), lambda i: (i, 0)),
            pl.BlockSpec((r, d), lambda i: (i, 0)),
            pl.BlockSpec((r, d), lambda i: (i, 0)),
            pl.BlockSpec((d, d), lambda i: (0, 0)),
            pl.BlockSpec((1, d), lambda i: (0, 0)),
            pl.BlockSpec((d, d), lambda i: (0, 0)),
            pl.BlockSpec((1, d), lambda i: (0, 0)),
        ],
        out_specs=pl.BlockSpec((r, d), lambda i: (i, 0)),
        out_shape=jax.ShapeDtypeStruct((n, d), jnp.float32),
    )(x, p0, p1, W1, b1.reshape(1, d), W2, b2.reshape(1, d))


def kernel(x, edge_index, W1, b1, W2, b2):
    n, d = x.shape
    e = edge_index.shape[1]

    rps = -(-(n + 1) // _NS)
    rps = -(-rps // 8) * 8
    n_pad = _NS * rps

    blocks_pad = -(-e // (_K * _NS)) * _NS
    e_pad = blocks_pad * _K
    bpw_tot = blocks_pad // _NS
    bpw0 = max(1, round(bpw_tot * _SPLIT0))
    bpw1 = bpw_tot - bpw0
    src = edge_index[0]
    dst = edge_index[1]
    if e_pad != e:
        pad = e_pad - e
        src = jnp.concatenate([src, jnp.zeros((pad,), jnp.int32)])
        dst = jnp.concatenate([dst, jnp.full((pad,), n, jnp.int32)])
    src = src.reshape(blocks_pad, 1, _K)
    dst = dst.reshape(blocks_pad, 1, _K)

    zeros = jnp.zeros((rps, d), jnp.float32)
    partials = _sc_aggregate(x, src, dst, zeros, n_pad=n_pad, rps=rps,
                             bpw0=bpw0, bpw1=bpw1)
    p0 = partials[:n]
    p1 = partials[n_pad:n_pad + n]
    return _mlp(x, p0, p1, W1, b1, W2, b2)


# speedup vs baseline: 1.1834x; 1.0456x over previous
import functools

import jax
import jax.numpy as jnp
from jax import lax
from jax.experimental import pallas as pl
from jax.experimental.pallas import tpu as pltpu
from jax.experimental.pallas import tpu_sc as plsc

_NC = 2
_NS = 16
_K = 128
_SPLIT0 = 0.45


def _sc_aggregate(x, src, dst, zeros, *, n_pad, rps, bpw0, bpw1):
    d = x.shape[1]
    mesh = plsc.VectorSubcoreMesh(core_axis_name="c", subcore_axis_name="s")

    seg_max = 80

    @functools.partial(
        pl.kernel,
        out_type=jax.ShapeDtypeStruct((_NC * n_pad, d), jnp.float32),
        mesh=mesh,
        scratch_types=[
            pltpu.VMEM((seg_max, 1, _K), jnp.int32),
            pltpu.VMEM((seg_max, 1, _K), jnp.int32),
            pltpu.VMEM((_K, d), jnp.float32),
            pltpu.VMEM_SHARED((n_pad, d), jnp.float32),
        ],
    )
    def agg_kernel(x_hbm, src_hbm, dst_hbm, z_hbm, out_hbm,
                   sidx, didx, rows, acc):
        cid = lax.axis_index("c")
        sid = lax.axis_index("s")

        pltpu.sync_copy(z_hbm, acc.at[pl.ds(sid * rps, rps)])
        plsc.subcore_barrier()

        def emit_worker(base, nblk):
            off = 0
            while off < nblk:
                seg = min(seg_max, nblk - off)
                pltpu.sync_copy(src_hbm.at[pl.ds(base + off, seg)],
                                sidx.at[pl.ds(0, seg)])
                pltpu.sync_copy(dst_hbm.at[pl.ds(base + off, seg)],
                                didx.at[pl.ds(0, seg)])
                for j in range(seg):
                    pltpu.sync_copy(x_hbm.at[sidx.at[j, 0]], rows)
                    pltpu.sync_copy(rows, acc.at[didx.at[j, 0]], add=True)
                off += seg

        @pl.when(cid == 0)
        def _():
            emit_worker(sid * bpw0, bpw0)

        @pl.when(cid == 1)
        def _():
            emit_worker(_NS * bpw0 + sid * bpw1, bpw1)

        plsc.subcore_barrier()
        pltpu.sync_copy(
            acc.at[pl.ds(sid * rps, rps)],
            out_hbm.at[pl.ds(cid * n_pad + sid * rps, rps)],
        )

    return agg_kernel(x, src, dst, zeros)


def _mlp(x, p0, p1, W1, b1, W2, b2):
    n, d = x.shape
    r = 1000
    assert n % r == 0

    def body(x_ref, p0_ref, p1_ref, w1_ref, b1_ref, w2_ref, b2_ref, o_ref):
        h = x_ref[...] + p0_ref[...] + p1_ref[...]
        h = jnp.dot(h, w1_ref[...], preferred_element_type=jnp.float32)
        h = jnp.maximum(h + b1_ref[...], 0.0)
        o = jnp.dot(h, w2_ref[...], preferred_element_type=jnp.float32)
        o_ref[...] = o + b2_ref[...]

    return pl.pallas_call(
        body,
        grid=(n // r,),
        in_specs=[
            pl.BlockSpec((r, d), lambda i: (i, 0)),
            pl.BlockSpec((r, d), lambda i: (i, 0)),
            pl.BlockSpec((r, d), lambda i: (i, 0)),
            pl.BlockSpec((d, d), lambda i: (0, 0)),
            pl.BlockSpec((1, d), lambda i: (0, 0)),
            pl.BlockSpec((d, d), lambda i: (0, 0)),
            pl.BlockSpec((1, d), lambda i: (0, 0)),
        ],
        out_specs=pl.BlockSpec((r, d), lambda i: (i, 0)),
        out_shape=jax.ShapeDtypeStruct((n, d), jnp.float32),
    )(x, p0, p1, W1, b1.reshape(1, d), W2, b2.reshape(1, d))


def kernel(x, edge_index, W1, b1, W2, b2):
    n, d = x.shape
    e = edge_index.shape[1]

    rps = -(-(n + 1) // _NS)
    rps = -(-rps // 8) * 8
    n_pad = _NS * rps

    blocks_pad = -(-e // (_K * _NS)) * _NS
    e_pad = blocks_pad * _K
    bpw_tot = blocks_pad // _NS
    bpw0 = max(1, round(bpw_tot * _SPLIT0))
    bpw1 = bpw_tot - bpw0
    src = edge_index[0]
    dst = edge_index[1]
    if e_pad != e:
        pad = e_pad - e
        src = jnp.concatenate([src, jnp.zeros((pad,), jnp.int32)])
        dst = jnp.concatenate([dst, jnp.full((pad,), n, jnp.int32)])
    src = src.reshape(blocks_pad, 1, _K)
    dst = dst.reshape(blocks_pad, 1, _K)

    zeros = jnp.zeros((rps, d), jnp.float32)
    partials = _sc_aggregate(x, src, dst, zeros, n_pad=n_pad, rps=rps,
                             bpw0=bpw0, bpw1=bpw1)
    p0 = partials[:n]
    p1 = partials[n_pad:n_pad + n]
    return _mlp(x, p0, p1, W1, b1, W2, b2)


# speedup vs baseline: 1.2097x; 1.0222x over previous
import functools

import jax
import jax.numpy as jnp
from jax import lax
from jax.experimental import pallas as pl
from jax.experimental.pallas import tpu as pltpu
from jax.experimental.pallas import tpu_sc as plsc

_NC = 2
_NS = 16
_K = 128
_SPLIT0 = 0.48


def _sc_aggregate(x, src, dst, zeros, *, n_pad, rps, bpw0, bpw1):
    d = x.shape[1]
    mesh = plsc.VectorSubcoreMesh(core_axis_name="c", subcore_axis_name="s")

    seg_max = 80

    @functools.partial(
        pl.kernel,
        out_type=jax.ShapeDtypeStruct((_NC * n_pad, d), jnp.float32),
        mesh=mesh,
        scratch_types=[
            pltpu.VMEM((seg_max, 1, _K), jnp.int32),
            pltpu.VMEM((seg_max, 1, _K), jnp.int32),
            pltpu.VMEM((_K, d), jnp.float32),
            pltpu.VMEM_SHARED((n_pad, d), jnp.float32),
        ],
    )
    def agg_kernel(x_hbm, src_hbm, dst_hbm, z_hbm, out_hbm,
                   sidx, didx, rows, acc):
        cid = lax.axis_index("c")
        sid = lax.axis_index("s")

        pltpu.sync_copy(z_hbm, acc.at[pl.ds(sid * rps, rps)])
        plsc.subcore_barrier()

        def emit_worker(base, nblk):
            off = 0
            while off < nblk:
                seg = min(seg_max, nblk - off)
                pltpu.sync_copy(src_hbm.at[pl.ds(base + off, seg)],
                                sidx.at[pl.ds(0, seg)])
                pltpu.sync_copy(dst_hbm.at[pl.ds(base + off, seg)],
                                didx.at[pl.ds(0, seg)])
                for j in range(seg):
                    pltpu.sync_copy(x_hbm.at[sidx.at[j, 0]], rows)
                    pltpu.sync_copy(rows, acc.at[didx.at[j, 0]], add=True)
                off += seg

        @pl.when(cid == 0)
        def _():
            emit_worker(sid * bpw0, bpw0)

        @pl.when(cid == 1)
        def _():
            emit_worker(_NS * bpw0 + sid * bpw1, bpw1)

        plsc.subcore_barrier()
        pltpu.sync_copy(
            acc.at[pl.ds(sid * rps, rps)],
            out_hbm.at[pl.ds(cid * n_pad + sid * rps, rps)],
        )

    return agg_kernel(x, src, dst, zeros)


def _mlp(x, p0, p1, W1, b1, W2, b2):
    n, d = x.shape
    r = 1000
    assert n % r == 0

    def body(x_ref, p0_ref, p1_ref, w1_ref, b1_ref, w2_ref, b2_ref, o_ref):
        h = x_ref[...] + p0_ref[...] + p1_ref[...]
        h = jnp.dot(h, w1_ref[...], preferred_element_type=jnp.float32)
        h = jnp.maximum(h + b1_ref[...], 0.0)
        o = jnp.dot(h, w2_ref[...], preferred_element_type=jnp.float32)
        o_ref[...] = o + b2_ref[...]

    return pl.pallas_call(
        body,
        grid=(n // r,),
        in_specs=[
            pl.BlockSpec((r, d), lambda i: (i, 0)),
            pl.BlockSpec((r, d), lambda i: (i, 0)),
            pl.BlockSpec((r, d), lambda i: (i, 0)),
            pl.BlockSpec((d, d), lambda i: (0, 0)),
            pl.BlockSpec((1, d), lambda i: (0, 0)),
            pl.BlockSpec((d, d), lambda i: (0, 0)),
            pl.BlockSpec((1, d), lambda i: (0, 0)),
        ],
        out_specs=pl.BlockSpec((r, d), lambda i: (i, 0)),
        out_shape=jax.ShapeDtypeStruct((n, d), jnp.float32),
    )(x, p0, p1, W1, b1.reshape(1, d), W2, b2.reshape(1, d))


def kernel(x, edge_index, W1, b1, W2, b2):
    n, d = x.shape
    e = edge_index.shape[1]

    rps = -(-(n + 1) // _NS)
    rps = -(-rps // 8) * 8
    n_pad = _NS * rps

    blocks_pad = -(-e // (_K * _NS)) * _NS
    e_pad = blocks_pad * _K
    bpw_tot = blocks_pad // _NS
    bpw0 = max(1, round(bpw_tot * _SPLIT0))
    bpw1 = bpw_tot - bpw0
    src = edge_index[0]
    dst = edge_index[1]
    if e_pad != e:
        pad = e_pad - e
        src = jnp.concatenate([src, jnp.zeros((pad,), jnp.int32)])
        dst = jnp.concatenate([dst, jnp.full((pad,), n, jnp.int32)])
    src = src.reshape(blocks_pad, 1, _K)
    dst = dst.reshape(blocks_pad, 1, _K)

    zeros = jnp.zeros((rps, d), jnp.float32)
    partials = _sc_aggregate(x, src, dst, zeros, n_pad=n_pad, rps=rps,
                             bpw0=bpw0, bpw1=bpw1)
    p0 = partials[:n]
    p1 = partials[n_pad:n_pad + n]
    return _mlp(x, p0, p1, W1, b1, W2, b2)


# speedup vs baseline: 1.2394x; 1.0245x over previous
import functools

import jax
import jax.numpy as jnp
from jax import lax
from jax.experimental import pallas as pl
from jax.experimental.pallas import tpu as pltpu
from jax.experimental.pallas import tpu_sc as plsc

_NC = 2
_NS = 16
_K = 128
_SPLIT0 = 0.50


def _sc_aggregate(x, src, dst, zeros, *, n_pad, rps, bpw0, bpw1):
    d = x.shape[1]
    mesh = plsc.VectorSubcoreMesh(core_axis_name="c", subcore_axis_name="s")

    seg_max = 80

    @functools.partial(
        pl.kernel,
        out_type=jax.ShapeDtypeStruct((_NC * n_pad, d), jnp.float32),
        mesh=mesh,
        scratch_types=[
            pltpu.VMEM((seg_max, 1, _K), jnp.int32),
            pltpu.VMEM((seg_max, 1, _K), jnp.int32),
            pltpu.VMEM((_K, d), jnp.float32),
            pltpu.VMEM_SHARED((n_pad, d), jnp.float32),
        ],
    )
    def agg_kernel(x_hbm, src_hbm, dst_hbm, z_hbm, out_hbm,
                   sidx, didx, rows, acc):
        cid = lax.axis_index("c")
        sid = lax.axis_index("s")

        pltpu.sync_copy(z_hbm, acc.at[pl.ds(sid * rps, rps)])
        plsc.subcore_barrier()

        def emit_worker(base, nblk):
            off = 0
            while off < nblk:
                seg = min(seg_max, nblk - off)
                pltpu.sync_copy(src_hbm.at[pl.ds(base + off, seg)],
                                sidx.at[pl.ds(0, seg)])
                pltpu.sync_copy(dst_hbm.at[pl.ds(base + off, seg)],
                                didx.at[pl.ds(0, seg)])
                for j in range(seg):
                    pltpu.sync_copy(x_hbm.at[sidx.at[j, 0]], rows)
                    pltpu.sync_copy(rows, acc.at[didx.at[j, 0]], add=True)
                off += seg

        @pl.when(cid == 0)
        def _():
            emit_worker(sid * bpw0, bpw0)

        @pl.when(cid == 1)
        def _():
            emit_worker(_NS * bpw0 + sid * bpw1, bpw1)

        plsc.subcore_barrier()
        pltpu.sync_copy(
            acc.at[pl.ds(sid * rps, rps)],
            out_hbm.at[pl.ds(cid * n_pad + sid * rps, rps)],
        )

    return agg_kernel(x, src, dst, zeros)


def _mlp(x, p0, p1, W1, b1, W2, b2):
    n, d = x.shape
    r = 1000
    assert n % r == 0

    def body(x_ref, p0_ref, p1_ref, w1_ref, b1_ref, w2_ref, b2_ref, o_ref):
        h = x_ref[...] + p0_ref[...] + p1_ref[...]
        h = jnp.dot(h, w1_ref[...], preferred_element_type=jnp.float32)
        h = jnp.maximum(h + b1_ref[...], 0.0)
        o = jnp.dot(h, w2_ref[...], preferred_element_type=jnp.float32)
        o_ref[...] = o + b2_ref[...]

    return pl.pallas_call(
        body,
        grid=(n // r,),
        in_specs=[
            pl.BlockSpec((r, d), lambda i: (i, 0)),
            pl.BlockSpec((r, d), lambda i: (i, 0)),
            pl.BlockSpec((r, d), lambda i: (i, 0)),
            pl.BlockSpec((d, d), lambda i: (0, 0)),
            pl.BlockSpec((1, d), lambda i: (0, 0)),
            pl.BlockSpec((d, d), lambda i: (0, 0)),
            pl.BlockSpec((1, d), lambda i: (0, 0)),
        ],
        out_specs=pl.BlockSpec((r, d), lambda i: (i, 0)),
        out_shape=jax.ShapeDtypeStruct((n, d), jnp.float32),
    )(x, p0, p1, W1, b1.reshape(1, d), W2, b2.reshape(1, d))


def kernel(x, edge_index, W1, b1, W2, b2):
    n, d = x.shape
    e = edge_index.shape[1]

    rps = -(-(n + 1) // _NS)
    rps = -(-rps // 8) * 8
    n_pad = _NS * rps

    blocks_pad = -(-e // (_K * _NS)) * _NS
    e_pad = blocks_pad * _K
    bpw_tot = blocks_pad // _NS
    bpw0 = max(1, round(bpw_tot * _SPLIT0))
    bpw1 = bpw_tot - bpw0
    src = edge_index[0]
    dst = edge_index[1]
    if e_pad != e:
        pad = e_pad - e
        src = jnp.concatenate([src, jnp.zeros((pad,), jnp.int32)])
        dst = jnp.concatenate([dst, jnp.full((pad,), n, jnp.int32)])
    src = src.reshape(blocks_pad, 1, _K)
    dst = dst.reshape(blocks_pad, 1, _K)

    zeros = jnp.zeros((rps, d), jnp.float32)
    partials = _sc_aggregate(x, src, dst, zeros, n_pad=n_pad, rps=rps,
                             bpw0=bpw0, bpw1=bpw1)
    p0 = partials[:n]
    p1 = partials[n_pad:n_pad + n]
    return _mlp(x, p0, p1, W1, b1, W2, b2)


# speedup vs baseline: 1.2761x; 1.0296x over previous
import functools

import jax
import jax.numpy as jnp
from jax import lax
from jax.experimental import pallas as pl
from jax.experimental.pallas import tpu as pltpu
from jax.experimental.pallas import tpu_sc as plsc

_NC = 2
_NS = 16
_K = 128
_SPLIT0 = 0.53


def _sc_aggregate(x, src, dst, zeros, *, n_pad, rps, bpw0, bpw1):
    d = x.shape[1]
    mesh = plsc.VectorSubcoreMesh(core_axis_name="c", subcore_axis_name="s")

    seg_max = 80

    @functools.partial(
        pl.kernel,
        out_type=jax.ShapeDtypeStruct((_NC * n_pad, d), jnp.float32),
        mesh=mesh,
        scratch_types=[
            pltpu.VMEM((seg_max, 1, _K), jnp.int32),
            pltpu.VMEM((seg_max, 1, _K), jnp.int32),
            pltpu.VMEM((_K, d), jnp.float32),
            pltpu.VMEM_SHARED((n_pad, d), jnp.float32),
        ],
    )
    def agg_kernel(x_hbm, src_hbm, dst_hbm, z_hbm, out_hbm,
                   sidx, didx, rows, acc):
        cid = lax.axis_index("c")
        sid = lax.axis_index("s")

        pltpu.sync_copy(z_hbm, acc.at[pl.ds(sid * rps, rps)])
        plsc.subcore_barrier()

        def emit_worker(base, nblk):
            off = 0
            while off < nblk:
                seg = min(seg_max, nblk - off)
                pltpu.sync_copy(src_hbm.at[pl.ds(base + off, seg)],
                                sidx.at[pl.ds(0, seg)])
                pltpu.sync_copy(dst_hbm.at[pl.ds(base + off, seg)],
                                didx.at[pl.ds(0, seg)])
                for j in range(seg):
                    pltpu.sync_copy(x_hbm.at[sidx.at[j, 0]], rows)
                    pltpu.sync_copy(rows, acc.at[didx.at[j, 0]], add=True)
                off += seg

        @pl.when(cid == 0)
        def _():
            emit_worker(sid * bpw0, bpw0)

        @pl.when(cid == 1)
        def _():
            emit_worker(_NS * bpw0 + sid * bpw1, bpw1)

        plsc.subcore_barrier()
        pltpu.sync_copy(
            acc.at[pl.ds(sid * rps, rps)],
            out_hbm.at[pl.ds(cid * n_pad + sid * rps, rps)],
        )

    return agg_kernel(x, src, dst, zeros)


def _mlp(x, p0, p1, W1, b1, W2, b2):
    n, d = x.shape
    r = 1000
    assert n % r == 0

    def body(x_ref, p0_ref, p1_ref, w1_ref, b1_ref, w2_ref, b2_ref, o_ref):
        h = x_ref[...] + p0_ref[...] + p1_ref[...]
        h = jnp.dot(h, w1_ref[...], preferred_element_type=jnp.float32)
        h = jnp.maximum(h + b1_ref[...], 0.0)
        o = jnp.dot(h, w2_ref[...], preferred_element_type=jnp.float32)
        o_ref[...] = o + b2_ref[...]

    return pl.pallas_call(
        body,
        grid=(n // r,),
        in_specs=[
            pl.BlockSpec((r, d), lambda i: (i, 0)),
            pl.BlockSpec((r, d), lambda i: (i, 0)),
            pl.BlockSpec((r, d), lambda i: (i, 0)),
            pl.BlockSpec((d, d), lambda i: (0, 0)),
            pl.BlockSpec((1, d), lambda i: (0, 0)),
            pl.BlockSpec((d, d), lambda i: (0, 0)),
            pl.BlockSpec((1, d), lambda i: (0, 0)),
        ],
        out_specs=pl.BlockSpec((r, d), lambda i: (i, 0)),
        out_shape=jax.ShapeDtypeStruct((n, d), jnp.float32),
    )(x, p0, p1, W1, b1.reshape(1, d), W2, b2.reshape(1, d))


def kernel(x, edge_index, W1, b1, W2, b2):
    n, d = x.shape
    e = edge_index.shape[1]

    rps = -(-(n + 1) // _NS)
    rps = -(-rps // 8) * 8
    n_pad = _NS * rps

    blocks_pad = -(-e // (_K * _NS)) * _NS
    e_pad = blocks_pad * _K
    bpw_tot = blocks_pad // _NS
    bpw0 = max(1, round(bpw_tot * _SPLIT0))
    bpw1 = bpw_tot - bpw0
    src = edge_index[0]
    dst = edge_index[1]
    if e_pad != e:
        pad = e_pad - e
        src = jnp.concatenate([src, jnp.zeros((pad,), jnp.int32)])
        dst = jnp.concatenate([dst, jnp.full((pad,), n, jnp.int32)])
    src = src.reshape(blocks_pad, 1, _K)
    dst = dst.reshape(blocks_pad, 1, _K)

    zeros = jnp.zeros((rps, d), jnp.float32)
    partials = _sc_aggregate(x, src, dst, zeros, n_pad=n_pad, rps=rps,
                             bpw0=bpw0, bpw1=bpw1)
    p0 = partials[:n]
    p1 = partials[n_pad:n_pad + n]
    return _mlp(x, p0, p1, W1, b1, W2, b2)


# speedup vs baseline: 1.3170x; 1.0320x over previous
import functools

import jax
import jax.numpy as jnp
from jax import lax
from jax.experimental import pallas as pl
from jax.experimental.pallas import tpu as pltpu
from jax.experimental.pallas import tpu_sc as plsc

_NC = 2
_NS = 16
_K = 128
_SPLIT0 = 0.56


def _sc_aggregate(x, src, dst, zeros, *, n_pad, rps, bpw0, bpw1):
    d = x.shape[1]
    mesh = plsc.VectorSubcoreMesh(core_axis_name="c", subcore_axis_name="s")

    seg_max = 80

    @functools.partial(
        pl.kernel,
        out_type=jax.ShapeDtypeStruct((_NC * n_pad, d), jnp.float32),
        mesh=mesh,
        scratch_types=[
            pltpu.VMEM((seg_max, 1, _K), jnp.int32),
            pltpu.VMEM((seg_max, 1, _K), jnp.int32),
            pltpu.VMEM((_K, d), jnp.float32),
            pltpu.VMEM_SHARED((n_pad, d), jnp.float32),
        ],
    )
    def agg_kernel(x_hbm, src_hbm, dst_hbm, z_hbm, out_hbm,
                   sidx, didx, rows, acc):
        cid = lax.axis_index("c")
        sid = lax.axis_index("s")

        pltpu.sync_copy(z_hbm, acc.at[pl.ds(sid * rps, rps)])
        plsc.subcore_barrier()

        def emit_worker(base, nblk):
            off = 0
            while off < nblk:
                seg = min(seg_max, nblk - off)
                pltpu.sync_copy(src_hbm.at[pl.ds(base + off, seg)],
                                sidx.at[pl.ds(0, seg)])
                pltpu.sync_copy(dst_hbm.at[pl.ds(base + off, seg)],
                                didx.at[pl.ds(0, seg)])
                for j in range(seg):
                    pltpu.sync_copy(x_hbm.at[sidx.at[j, 0]], rows)
                    pltpu.sync_copy(rows, acc.at[didx.at[j, 0]], add=True)
                off += seg

        @pl.when(cid == 0)
        def _():
            emit_worker(sid * bpw0, bpw0)

        @pl.when(cid == 1)
        def _():
            emit_worker(_NS * bpw0 + sid * bpw1, bpw1)

        plsc.subcore_barrier()
        pltpu.sync_copy(
            acc.at[pl.ds(sid * rps, rps)],
            out_hbm.at[pl.ds(cid * n_pad + sid * rps, rps)],
        )

    return agg_kernel(x, src, dst, zeros)


def _mlp(x, p0, p1, W1, b1, W2, b2):
    n, d = x.shape
    r = 1000
    assert n % r == 0

    def body(x_ref, p0_ref, p1_ref, w1_ref, b1_ref, w2_ref, b2_ref, o_ref):
        h = x_ref[...] + p0_ref[...] + p1_ref[...]
        h = jnp.dot(h, w1_ref[...], preferred_element_type=jnp.float32)
        h = jnp.maximum(h + b1_ref[...], 0.0)
        o = jnp.dot(h, w2_ref[...], preferred_element_type=jnp.float32)
        o_ref[...] = o + b2_ref[...]

    return pl.pallas_call(
        body,
        grid=(n // r,),
        in_specs=[
            pl.BlockSpec((r, d), lambda i: (i, 0)),
            pl.BlockSpec((r, d), lambda i: (i, 0)),
            pl.BlockSpec((r, d), lambda i: (i, 0)),
            pl.BlockSpec((d, d), lambda i: (0, 0)),
            pl.BlockSpec((1, d), lambda i: (0, 0)),
            pl.BlockSpec((d, d), lambda i: (0, 0)),
            pl.BlockSpec((1, d), lambda i: (0, 0)),
        ],
        out_specs=pl.BlockSpec((r, d), lambda i: (i, 0)),
        out_shape=jax.ShapeDtypeStruct((n, d), jnp.float32),
    )(x, p0, p1, W1, b1.reshape(1, d), W2, b2.reshape(1, d))


def kernel(x, edge_index, W1, b1, W2, b2):
    n, d = x.shape
    e = edge_index.shape[1]

    rps = -(-(n + 1) // _NS)
    rps = -(-rps // 8) * 8
    n_pad = _NS * rps

    blocks_pad = -(-e // (_K * _NS)) * _NS
    e_pad = blocks_pad * _K
    bpw_tot = blocks_pad // _NS
    bpw0 = max(1, round(bpw_tot * _SPLIT0))
    bpw1 = bpw_tot - bpw0
    src = edge_index[0]
    dst = edge_index[1]
    if e_pad != e:
        pad = e_pad - e
        src = jnp.concatenate([src, jnp.zeros((pad,), jnp.int32)])
        dst = jnp.concatenate([dst, jnp.full((pad,), n, jnp.int32)])
    src = src.reshape(blocks_pad, 1, _K)
    dst = dst.reshape(blocks_pad, 1, _K)

    zeros = jnp.zeros((rps, d), jnp.float32)
    partials = _sc_aggregate(x, src, dst, zeros, n_pad=n_pad, rps=rps,
                             bpw0=bpw0, bpw1=bpw1)
    p0 = partials[:n]
    p1 = partials[n_pad:n_pad + n]
    return _mlp(x, p0, p1, W1, b1, W2, b2)


# speedup vs baseline: 1.3730x; 1.0425x over previous
import functools

import jax
import jax.numpy as jnp
from jax import lax
from jax.experimental import pallas as pl
from jax.experimental.pallas import tpu as pltpu
from jax.experimental.pallas import tpu_sc as plsc

_NC = 2
_NS = 16
_K = 128
_SPLIT0 = 0.60


def _sc_aggregate(x, src, dst, zeros, *, n_pad, rps, bpw0, bpw1):
    d = x.shape[1]
    mesh = plsc.VectorSubcoreMesh(core_axis_name="c", subcore_axis_name="s")

    seg_max = 80

    @functools.partial(
        pl.kernel,
        out_type=jax.ShapeDtypeStruct((_NC * n_pad, d), jnp.float32),
        mesh=mesh,
        scratch_types=[
            pltpu.VMEM((seg_max, 1, _K), jnp.int32),
            pltpu.VMEM((seg_max, 1, _K), jnp.int32),
            pltpu.VMEM((_K, d), jnp.float32),
            pltpu.VMEM_SHARED((n_pad, d), jnp.float32),
        ],
    )
    def agg_kernel(x_hbm, src_hbm, dst_hbm, z_hbm, out_hbm,
                   sidx, didx, rows, acc):
        cid = lax.axis_index("c")
        sid = lax.axis_index("s")

        pltpu.sync_copy(z_hbm, acc.at[pl.ds(sid * rps, rps)])
        plsc.subcore_barrier()

        def emit_worker(base, nblk):
            off = 0
            while off < nblk:
                seg = min(seg_max, nblk - off)
                pltpu.sync_copy(src_hbm.at[pl.ds(base + off, seg)],
                                sidx.at[pl.ds(0, seg)])
                pltpu.sync_copy(dst_hbm.at[pl.ds(base + off, seg)],
                                didx.at[pl.ds(0, seg)])
                for j in range(seg):
                    pltpu.sync_copy(x_hbm.at[sidx.at[j, 0]], rows)
                    pltpu.sync_copy(rows, acc.at[didx.at[j, 0]], add=True)
                off += seg

        @pl.when(cid == 0)
        def _():
            emit_worker(sid * bpw0, bpw0)

        @pl.when(cid == 1)
        def _():
            emit_worker(_NS * bpw0 + sid * bpw1, bpw1)

        plsc.subcore_barrier()
        pltpu.sync_copy(
            acc.at[pl.ds(sid * rps, rps)],
            out_hbm.at[pl.ds(cid * n_pad + sid * rps, rps)],
        )

    return agg_kernel(x, src, dst, zeros)


def _mlp(x, p0, p1, W1, b1, W2, b2):
    n, d = x.shape
    r = 1000
    assert n % r == 0

    def body(x_ref, p0_ref, p1_ref, w1_ref, b1_ref, w2_ref, b2_ref, o_ref):
        h = x_ref[...] + p0_ref[...] + p1_ref[...]
        h = jnp.dot(h, w1_ref[...], preferred_element_type=jnp.float32)
        h = jnp.maximum(h + b1_ref[...], 0.0)
        o = jnp.dot(h, w2_ref[...], preferred_element_type=jnp.float32)
        o_ref[...] = o + b2_ref[...]

    return pl.pallas_call(
        body,
        grid=(n // r,),
        in_specs=[
            pl.BlockSpec((r, d), lambda i: (i, 0)),
            pl.BlockSpec((r, d), lambda i: (i, 0)),
            pl.BlockSpec((r, d), lambda i: (i, 0)),
            pl.BlockSpec((d, d), lambda i: (0, 0)),
            pl.BlockSpec((1, d), lambda i: (0, 0)),
            pl.BlockSpec((d, d), lambda i: (0, 0)),
            pl.BlockSpec((1, d), lambda i: (0, 0)),
        ],
        out_specs=pl.BlockSpec((r, d), lambda i: (i, 0)),
        out_shape=jax.ShapeDtypeStruct((n, d), jnp.float32),
    )(x, p0, p1, W1, b1.reshape(1, d), W2, b2.reshape(1, d))


def kernel(x, edge_index, W1, b1, W2, b2):
    n, d = x.shape
    e = edge_index.shape[1]

    rps = -(-(n + 1) // _NS)
    rps = -(-rps // 8) * 8
    n_pad = _NS * rps

    blocks_pad = -(-e // (_K * _NS)) * _NS
    e_pad = blocks_pad * _K
    bpw_tot = blocks_pad // _NS
    bpw0 = max(1, round(bpw_tot * _SPLIT0))
    bpw1 = bpw_tot - bpw0
    src = edge_index[0]
    dst = edge_index[1]
    if e_pad != e:
        pad = e_pad - e
        src = jnp.concatenate([src, jnp.zeros((pad,), jnp.int32)])
        dst = jnp.concatenate([dst, jnp.full((pad,), n, jnp.int32)])
    src = src.reshape(blocks_pad, 1, _K)
    dst = dst.reshape(blocks_pad, 1, _K)

    zeros = jnp.zeros((rps, d), jnp.float32)
    partials = _sc_aggregate(x, src, dst, zeros, n_pad=n_pad, rps=rps,
                             bpw0=bpw0, bpw1=bpw1)
    p0 = partials[:n]
    p1 = partials[n_pad:n_pad + n]
    return _mlp(x, p0, p1, W1, b1, W2, b2)
